# baseline (speedup 1.0000x reference)
import functools

import jax
import jax.numpy as jnp
from jax import lax
from jax.experimental import pallas as pl
from jax.experimental.pallas import tpu as pltpu
from jax.experimental.pallas import tpu_sc as plsc

B = 16384
H = 50
D = 64
N = B * H
NC = 2
NS = 16
NW = NC * NS
BPW = B // NW
PER_W = BPW * H
NBT = BPW // 128
NCHUNK = H * NBT
CL = 128 * D


def _gather_kernel(idx_hbm, table_hbm, out_hbm,
                   idx_v, cidx_v, g_v, ob_v, sem_g0, sem_g1, sem_w0, sem_w1):
    wid = lax.axis_index("s") * NC + lax.axis_index("c")
    b0 = wid * BPW
    pltpu.sync_copy(idx_hbm.at[pl.ds(b0 * H, PER_W)], idx_v)

    sem_g = (sem_g0, sem_g1)
    sem_w = (sem_w0, sem_w1)
    lanes = lax.iota(jnp.int32, 16)
    lanes_h = lanes * H

    def decode(i):
        return i // NBT, i % NBT

    def repack(i, s):
        h, t = decode(i)
        base = t * (128 * H) + h
        for k in range(8):
            vals = plsc.load_gather(idx_v, [lanes_h + (base + k * 16 * H)])
            cidx_v[s, pl.ds(k * 16, 16)] = vals

    def gather(i, s):
        return pltpu.make_async_copy(
            table_hbm.at[cidx_v.at[s]], g_v.at[s], sem_g[s])

    def writeback(i, s):
        h, t = decode(i)
        return pltpu.make_async_copy(
            ob_v.at[s], out_hbm.at[h, :, wid * NBT + t], sem_w[s])

    rows_k = [lanes + (k * 16) for k in range(8)]
    cols_c = [jnp.full((16,), c, jnp.int32) for c in range(D)]

    pairs = [(c, k) for c in range(D) for k in range(8)]

    def transpose(i, s):
        g_s = g_v.at[s]
        for p0 in range(0, len(pairs), 16):
            batch = pairs[p0:p0 + 16]
            vals = [plsc.load_gather(g_s, [rows_k[k], cols_c[c]])
                    for (c, k) in batch]
            for (c, k), v in zip(batch, vals):
                ob_v[s, c // 8, c % 8, pl.ds(k * 16, 16)] = v

    for s in range(2):
        repack(s, s)
        gather(s, s).start()

    def body(i, carry):
        for s in range(2):
            c = i * 2 + s
            gather(c, s).wait()

            @pl.when(c >= 2)
            def _():
                writeback(c - 2, s).wait()

            transpose(c, s)
            writeback(c, s).start()

            @pl.when(c + 2 < NCHUNK)
            def _():
                repack(c + 2, s)
                gather(c + 2, s).start()
        return carry

    lax.fori_loop(0, NCHUNK // 2, body, 0)

    for s in range(2):
        writeback(NCHUNK - 2 + s, s).wait()


@jax.jit
def _lookup(idx_flat, table):
    mesh = plsc.VectorSubcoreMesh(
        core_axis_name="c", subcore_axis_name="s",
        num_cores=NC, num_subcores=NS,
    )
    run = functools.partial(
        pl.kernel,
        out_type=jax.ShapeDtypeStruct((H, 8, 128, 8, 128), jnp.float32),
        mesh=mesh,
        scratch_types=[
            pltpu.VMEM((PER_W,), jnp.int32),
            pltpu.VMEM((2, 128), jnp.int32),
            pltpu.VMEM((2, 128, D), jnp.float32),
            pltpu.VMEM((2, 8, 8, 128), jnp.float32),
            pltpu.SemaphoreType.DMA,
            pltpu.SemaphoreType.DMA,
            pltpu.SemaphoreType.DMA,
            pltpu.SemaphoreType.DMA,
        ],
        compiler_params=pltpu.CompilerParams(
            use_tc_tiling_on_sc=False,
            needs_layout_passes=False,
            disable_bounds_checks=True,
        ),
    )(_gather_kernel)
    return run(idx_flat, table)


def kernel(item_idx, table):
    idx_flat = item_idx.reshape(N).astype(jnp.int32)
    out_t = _lookup(idx_flat, table)
    return out_t.transpose(2, 4, 0, 1, 3).reshape(B, H, D)


# speedup vs baseline: 1.9566x; 1.9566x over previous
import functools

import jax
import jax.numpy as jnp
from jax import lax
from jax.experimental import pallas as pl
from jax.experimental.pallas import tpu as pltpu
from jax.experimental.pallas import tpu_sc as plsc

B = 16384
H = 50
D = 64
N = B * H
NC = 2
NS = 16
NW = NC * NS
BPW = B // NW
PER_W = BPW * H
NBT = BPW // 128
NCHUNK = H * NBT
CL = 128 * D


def _gather_kernel(idx_hbm, table_hbm, out_hbm,
                   idx_v, cidx_v, g_v, ob_v, sem_g0, sem_g1, sem_w0, sem_w1):
    wid = lax.axis_index("s") * NC + lax.axis_index("c")
    b0 = wid * BPW
    pltpu.sync_copy(idx_hbm.at[pl.ds(b0 * H, PER_W)], idx_v)

    sem_g = (sem_g0, sem_g1)
    sem_w = (sem_w0, sem_w1)
    lanes = lax.iota(jnp.int32, 16)
    lanes_h = lanes * H

    def decode(i):
        return i // NBT, i % NBT

    def repack(i, s):
        h, t = decode(i)
        base = t * (128 * H) + h
        for k in range(8):
            vals = plsc.load_gather(idx_v, [lanes_h + (base + k * 16 * H)])
            cidx_v[s, pl.ds(k * 16, 16)] = vals

    def gather(i, s):
        return pltpu.make_async_copy(
            table_hbm.at[cidx_v.at[s]], g_v.at[s], sem_g[s])

    def writeback(i, s):
        h, t = decode(i)
        return pltpu.make_async_copy(
            ob_v.at[s], out_hbm.at[h, :, wid * NBT + t], sem_w[s])

    rot = [lax.rem(lanes + j, 16) for j in range(16)]

    def transpose(i, s):
        g_s = g_v.at[s]
        ob_s = ob_v.at[s]

        def block(kcg, carry):
            k = kcg // (D // 16)
            cb = lax.rem(kcg, D // 16) * 16
            rows = lanes + k * 16
            cols = [cb + rot[j] for j in range(16)]
            vals = [plsc.load_gather(g_s, [rows, cols[j]]) for j in range(16)]
            for j in range(16):
                plsc.store_scatter(
                    ob_s,
                    [lax.shift_right_logical(cols[j], 3),
                     lax.bitwise_and(cols[j], 7), rows],
                    vals[j])
            return carry

        lax.fori_loop(0, 8 * (D // 16), block, 0)

    for s in range(2):
        repack(s, s)
        gather(s, s).start()

    def body(i, carry):
        for s in range(2):
            c = i * 2 + s
            gather(c, s).wait()

            @pl.when(c >= 2)
            def _():
                writeback(c - 2, s).wait()

            transpose(c, s)
            writeback(c, s).start()

            @pl.when(c + 2 < NCHUNK)
            def _():
                repack(c + 2, s)
                gather(c + 2, s).start()
        return carry

    lax.fori_loop(0, NCHUNK // 2, body, 0)

    for s in range(2):
        writeback(NCHUNK - 2 + s, s).wait()


@jax.jit
def _lookup(idx_flat, table):
    mesh = plsc.VectorSubcoreMesh(
        core_axis_name="c", subcore_axis_name="s",
        num_cores=NC, num_subcores=NS,
    )
    run = functools.partial(
        pl.kernel,
        out_type=jax.ShapeDtypeStruct((H, 8, 128, 8, 128), jnp.float32),
        mesh=mesh,
        scratch_types=[
            pltpu.VMEM((PER_W,), jnp.int32),
            pltpu.VMEM((2, 128), jnp.int32),
            pltpu.VMEM((2, 128, D), jnp.float32),
            pltpu.VMEM((2, 8, 8, 128), jnp.float32),
            pltpu.SemaphoreType.DMA,
            pltpu.SemaphoreType.DMA,
            pltpu.SemaphoreType.DMA,
            pltpu.SemaphoreType.DMA,
        ],
        compiler_params=pltpu.CompilerParams(
            use_tc_tiling_on_sc=False,
            needs_layout_passes=False,
            disable_bounds_checks=True,
        ),
    )(_gather_kernel)
    return run(idx_flat, table)


def kernel(item_idx, table):
    idx_flat = item_idx.reshape(N).astype(jnp.int32)
    out_t = _lookup(idx_flat, table)
    return out_t.transpose(2, 4, 0, 1, 3).reshape(B, H, D)


# speedup vs baseline: 2.0198x; 1.0323x over previous
import functools

import jax
import jax.numpy as jnp
from jax import lax
from jax.experimental import pallas as pl
from jax.experimental.pallas import tpu as pltpu
from jax.experimental.pallas import tpu_sc as plsc

B = 16384
H = 50
D = 64
N = B * H
NC = 2
NS = 16
NW = NC * NS
BPW = B // NW
PER_W = BPW * H
NBT = BPW // 128
NCHUNK = H * NBT
CL = 128 * D


TAIL0 = 999936


def _gather_kernel(idx_hbm, table_hbm, tail_hbm, out_hbm,
                   idx_v, cidx_v, cidx2_v, tail_v, g_v, ob_v,
                   sem_g0, sem_g1, sem_w0, sem_w1):
    wid = lax.axis_index("s") * NC + lax.axis_index("c")
    b0 = wid * BPW
    pltpu.sync_copy(idx_hbm.at[pl.ds(b0 * H, PER_W)], idx_v)
    pltpu.sync_copy(tail_hbm, tail_v)

    sem_g = (sem_g0, sem_g1)
    sem_w = (sem_w0, sem_w1)
    lanes = lax.iota(jnp.int32, 16)
    lanes_h = lanes * H

    def decode(i):
        return i // NBT, i % NBT

    def repack(i, s):
        h, t = decode(i)
        base = t * (128 * H) + h
        for k in range(8):
            vals = plsc.load_gather(idx_v, [lanes_h + (base + k * 16 * H)])
            cidx_v[s, pl.ds(k * 16, 16)] = vals
            cidx2_v[s, pl.ds(k * 16, 16)] = jnp.minimum(vals, TAIL0 - 1)

    def gather(i, s):
        return pltpu.make_async_copy(
            table_hbm.at[cidx2_v.at[s]], g_v.at[s], sem_g[s])

    def writeback(i, s):
        h, t = decode(i)
        return pltpu.make_async_copy(
            ob_v.at[s], out_hbm.at[h, :, wid * NBT + t], sem_w[s])

    rot = [lax.rem(lanes + j, 16) for j in range(16)]

    def transpose(i, s):
        g_s = g_v.at[s]
        ob_s = ob_v.at[s]

        def block(kcg, carry):
            k = kcg // (D // 16)
            cb = lax.rem(kcg, D // 16) * 16
            rows = lanes + k * 16
            cols = [cb + rot[j] for j in range(16)]
            vals = [plsc.load_gather(g_s, [rows, cols[j]]) for j in range(16)]
            for j in range(16):
                plsc.store_scatter(
                    ob_s,
                    [lax.shift_right_logical(cols[j], 3),
                     lax.bitwise_and(cols[j], 7), rows],
                    vals[j])
            return carry

        lax.fori_loop(0, 8 * (D // 16), block, 0)

    def fixup_tail(s):
        g_s = g_v.at[s]
        mx0 = jnp.full((16,), 0, jnp.int32)

        def accmax(k, acc):
            ck = cidx_v[s, pl.ds(k * 16, 16)]
            return jnp.maximum(acc, ck)

        mx = lax.reduce_max(lax.fori_loop(0, 8, accmax, mx0), (0,))

        @pl.when(mx >= TAIL0)
        def _():
            for k in range(8):
                ck = cidx_v[s, pl.ds(k * 16, 16)]
                msk = ck >= TAIL0
                trow = jnp.maximum(ck - TAIL0, 0)
                rows = lanes + k * 16

                def fix_c(c, carry):
                    cfull = jnp.full((16,), 0, jnp.int32) + c
                    vals = plsc.load_gather(tail_v, [trow, cfull])
                    plsc.store_scatter(g_s, [rows, cfull], vals, mask=msk)
                    return carry

                lax.fori_loop(0, D, fix_c, 0)

    for s in range(2):
        repack(s, s)
        gather(s, s).start()

    def body(i, carry):
        for s in range(2):
            c = i * 2 + s
            gather(c, s).wait()

            @pl.when(c >= 2)
            def _():
                writeback(c - 2, s).wait()

            fixup_tail(s)
            transpose(c, s)
            writeback(c, s).start()

            @pl.when(c + 2 < NCHUNK)
            def _():
                repack(c + 2, s)
                gather(c + 2, s).start()
        return carry

    lax.fori_loop(0, NCHUNK // 2, body, 0)

    for s in range(2):
        writeback(NCHUNK - 2 + s, s).wait()


NLT = 999936 // 128
BASE_BLK = NLT // NW
EXTRA = NLT - BASE_BLK * NW


def _transpose_kernel(tt_hbm, out_hbm, tin_v0, tin_v1, tout_v0, tout_v1,
                      sem_i0, sem_i1, sem_o0, sem_o1):
    tin = (tin_v0, tin_v1)
    tout = (tout_v0, tout_v1)
    wid = lax.axis_index("s") * NC + lax.axis_index("c")
    nb = BASE_BLK + jnp.where(wid < EXTRA, 1, 0)
    start = wid * BASE_BLK + jnp.minimum(wid, EXTRA)

    sem_i = (sem_i0, sem_i1)
    sem_o = (sem_o0, sem_o1)
    lanes = lax.iota(jnp.int32, 16)
    rot = [lax.rem(lanes + j, 16) for j in range(16)]

    def load(j, s):
        return pltpu.make_async_copy(
            tt_hbm.at[:, pl.ds(j * 128, 128)], tin[s], sem_i[s])

    def store(j, s):
        return pltpu.make_async_copy(
            tout[s], out_hbm.at[pl.ds(j * 8192, 8192)], sem_o[s])

    def transpose(s):
        tin_s = tin[s]
        def block(rgcg, carry):
            r0 = (rgcg // 4) * 16
            cb = lax.rem(rgcg, 4) * 16
            cols = cb + lanes
            for j in range(16):
                rows = r0 + rot[j]
                vals = plsc.load_gather(tin_s, [cols, rows])
                plsc.store_scatter(tout[s], [rows * D + cols], vals)
            return carry
        lax.fori_loop(0, 32, block, 0)

    load(start, 0).start()
    load(start + 1, 1).start()

    def main_body(i, carry):
        for s in range(2):
            b = i * 2 + s
            load(start + b, s).wait()

            @pl.when(b >= 2)
            def _():
                store(start + b - 2, s).wait()

            transpose(s)
            store(start + b, s).start()

            @pl.when(b + 2 < nb)
            def _():
                load(start + b + 2, s).start()
        return carry

    lax.fori_loop(0, nb // 2, main_body, 0)

    @pl.when(lax.rem(nb, 2) == 1)
    def _():
        b = nb - 1
        load(start + b, 0).wait()
        store(start + b - 2, 0).wait()
        transpose(0)
        store(start + b, 0).start()

    for s in range(2):
        @pl.when(nb - 2 + s >= 0)
        def _():
            store(start + nb - 2 + s, s).wait()


@jax.jit
def _table_rowmajor(tableT):
    mesh = plsc.VectorSubcoreMesh(
        core_axis_name="c", subcore_axis_name="s",
        num_cores=NC, num_subcores=NS,
    )
    run = functools.partial(
        pl.kernel,
        out_type=jax.ShapeDtypeStruct((999936 * D,), jnp.float32),
        mesh=mesh,
        scratch_types=[
            pltpu.VMEM((D, 128), jnp.float32),
            pltpu.VMEM((D, 128), jnp.float32),
            pltpu.VMEM((128 * D,), jnp.float32),
            pltpu.VMEM((128 * D,), jnp.float32),
            pltpu.SemaphoreType.DMA,
            pltpu.SemaphoreType.DMA,
            pltpu.SemaphoreType.DMA,
            pltpu.SemaphoreType.DMA,
        ],
        compiler_params=pltpu.CompilerParams(
            use_tc_tiling_on_sc=True,
            needs_layout_passes=False,
            disable_bounds_checks=True,
        ),
    )(_transpose_kernel)
    return run(tableT)


@jax.jit
def _lookup(idx_flat, table, tail_rows):
    mesh = plsc.VectorSubcoreMesh(
        core_axis_name="c", subcore_axis_name="s",
        num_cores=NC, num_subcores=NS,
    )
    run = functools.partial(
        pl.kernel,
        out_type=jax.ShapeDtypeStruct((H, 8, 128, 8, 128), jnp.float32),
        mesh=mesh,
        scratch_types=[
            pltpu.VMEM((PER_W,), jnp.int32),
            pltpu.VMEM((2, 128), jnp.int32),
            pltpu.VMEM((2, 128), jnp.int32),
            pltpu.VMEM((64, D), jnp.float32),
            pltpu.VMEM((2, 128, D), jnp.float32),
            pltpu.VMEM((2, 8, 8, 128), jnp.float32),
            pltpu.SemaphoreType.DMA,
            pltpu.SemaphoreType.DMA,
            pltpu.SemaphoreType.DMA,
            pltpu.SemaphoreType.DMA,
        ],
        compiler_params=pltpu.CompilerParams(
            use_tc_tiling_on_sc=False,
            needs_layout_passes=False,
            disable_bounds_checks=True,
        ),
    )(_gather_kernel)
    return run(idx_flat, table, tail_rows)


def kernel(item_idx, table):
    idx_flat = item_idx.reshape(N).astype(jnp.int32)
    table_lin = _table_rowmajor(table.T).reshape(999936, D)
    tail_rows = table[999936:, :]
    out_t = _lookup(idx_flat, table_lin, tail_rows)
    return out_t.transpose(2, 4, 0, 1, 3).reshape(B, H, D)
